# Initial kernel scaffold; baseline (speedup 1.0000x reference)
import jax
import jax.numpy as jnp
from jax.experimental import pallas as pl


def kernel(adj, features, neighbors, W1, b1, W2, b2, W3, b3, Wfc, bfc, Wsc, bsc):
    raise NotImplementedError("write your pallas kernel here")


# speedup vs baseline: 13.0694x; 13.0694x over previous
import functools

import jax
import jax.numpy as jnp
from jax import lax
from jax.experimental import pallas as pl
from jax.experimental.pallas import tpu as pltpu
from jax.experimental.pallas import tpu_sc as plsc

N = 10000
NP = 10240
DUMMY = 10008
E = 320000
EP = 327680
CH = 128
PER_TILE = EP // 32
NCHUNK = PER_TILE // CH
NTILE = 16
RPT = NP // NTILE
D = 128
F1, F2, F3, BN, NC_ = 64, 32, 16, 8, 10
BLK = 1024

_MESH = plsc.VectorSubcoreMesh(core_axis_name="c", subcore_axis_name="s")
_SC_PARAMS = pltpu.CompilerParams(use_tc_tiling_on_sc=False)


@functools.partial(
    pl.kernel,
    mesh=_MESH,
    compiler_params=_SC_PARAMS,
    out_type=jax.ShapeDtypeStruct((2, NP), jnp.float32),
    scratch_types=[
        pltpu.VMEM((CH,), jnp.int32),
        pltpu.VMEM((CH,), jnp.float32),
        pltpu.VMEM_SHARED((NP,), jnp.float32),
    ],
)
def _deg_sc(dst_hbm, zero_hbm, out_hbm, idx_v, ones_v, acc_sh):
    c = lax.axis_index("c")
    s = lax.axis_index("s")
    wid = c * NTILE + s
    for j in range(CH // 16):
        ones_v[pl.ds(j * 16, 16)] = jnp.ones((16,), jnp.float32)
    pltpu.sync_copy(zero_hbm.at[pl.ds(s * RPT, RPT)],
                    acc_sh.at[pl.ds(s * RPT, RPT)])
    plsc.subcore_barrier()
    base = wid * PER_TILE

    def body(i, carry):
        pltpu.sync_copy(dst_hbm.at[pl.ds(base + i * CH, CH)], idx_v)
        pltpu.sync_copy(ones_v, acc_sh.at[idx_v], add=True)
        return carry

    lax.fori_loop(0, NCHUNK, body, 0)
    plsc.subcore_barrier()
    pltpu.sync_copy(acc_sh.at[pl.ds(s * RPT, RPT)],
                    out_hbm.at[c, pl.ds(s * RPT, RPT)])


def _make_agg_sc(F):
    @functools.partial(
        pl.kernel,
        mesh=_MESH,
        compiler_params=_SC_PARAMS,
        out_type=jax.ShapeDtypeStruct((2, NP, F), jnp.float32),
        scratch_types=[
            pltpu.VMEM((CH,), jnp.int32),
            pltpu.VMEM((CH,), jnp.int32),
            pltpu.VMEM((CH, F), jnp.float32),
            pltpu.VMEM_SHARED((NP, F), jnp.float32),
            pltpu.SemaphoreType.DMA,
        ],
    )
    def agg(g_hbm, zero_hbm, src_hbm, dst_hbm, out_hbm,
            src_v, dst_v, rows_v, acc_sh, sem):
        c = lax.axis_index("c")
        s = lax.axis_index("s")
        wid = c * NTILE + s
        pltpu.sync_copy(zero_hbm.at[pl.ds(s * RPT, RPT)],
                        acc_sh.at[pl.ds(s * RPT, RPT)])
        plsc.subcore_barrier()
        base = wid * PER_TILE

        def body(i, carry):
            off = base + i * CH
            pltpu.sync_copy(src_hbm.at[pl.ds(off, CH)], src_v)
            pltpu.sync_copy(dst_hbm.at[pl.ds(off, CH)], dst_v)
            pltpu.async_copy(g_hbm.at[src_v], rows_v, sem).wait()
            pltpu.sync_copy(rows_v, acc_sh.at[dst_v], add=True)
            return carry

        lax.fori_loop(0, NCHUNK, body, 0)
        plsc.subcore_barrier()
        pltpu.sync_copy(acc_sh.at[pl.ds(s * RPT, RPT)],
                        out_hbm.at[c, pl.ds(s * RPT, RPT)])

    return agg


_agg64 = _make_agg_sc(F1)
_agg32 = _make_agg_sc(F2)
_agg16 = _make_agg_sc(F3)


def _dinv_body(degT_ref, selfd_ref, o_ref):
    d = degT_ref[:, 0:1] + degT_ref[:, 1:2] + selfd_ref[...]
    o_ref[...] = jnp.where(d > 0, lax.rsqrt(jnp.maximum(d, 1e-12)),
                           jnp.zeros_like(d))


def _dinv_tc(degT, selfd):
    return pl.pallas_call(
        _dinv_body,
        grid=(NP // BLK,),
        in_specs=[pl.BlockSpec((BLK, 2), lambda i: (i, 0)),
                  pl.BlockSpec((BLK, 1), lambda i: (i, 0))],
        out_specs=pl.BlockSpec((BLK, 1), lambda i: (i, 0)),
        out_shape=jax.ShapeDtypeStruct((NP, 1), jnp.float32),
    )(degT, selfd)


def _mm1_body(x_ref, w_ref, dinv_ref, o_ref):
    h = jnp.dot(x_ref[...], w_ref[...], preferred_element_type=jnp.float32)
    o_ref[...] = h * dinv_ref[...]


def _mm1_tc(xp, W1, dinv):
    return pl.pallas_call(
        _mm1_body,
        grid=(NP // BLK,),
        in_specs=[pl.BlockSpec((BLK, D), lambda i: (i, 0)),
                  pl.BlockSpec((D, F1), lambda i: (0, 0)),
                  pl.BlockSpec((BLK, 1), lambda i: (i, 0))],
        out_specs=pl.BlockSpec((BLK, F1), lambda i: (i, 0)),
        out_shape=jax.ShapeDtypeStruct((NP, F1), jnp.float32),
    )(xp, W1, dinv)


def _layer_body(p_ref, g_ref, dinv_ref, b_ref, w_ref, o_ref):
    sfull = p_ref[0] + p_ref[1] + g_ref[...]
    h = jnp.maximum(sfull * dinv_ref[...] + b_ref[...], 0.0)
    o_ref[...] = jnp.dot(h, w_ref[...],
                         preferred_element_type=jnp.float32) * dinv_ref[...]


def _layer_tc(parts, g, dinv, b, W, F, Fn):
    return pl.pallas_call(
        _layer_body,
        grid=(NP // BLK,),
        in_specs=[pl.BlockSpec((2, BLK, F), lambda i: (0, i, 0)),
                  pl.BlockSpec((BLK, F), lambda i: (i, 0)),
                  pl.BlockSpec((BLK, 1), lambda i: (i, 0)),
                  pl.BlockSpec((1, F), lambda i: (0, 0)),
                  pl.BlockSpec((F, Fn), lambda i: (0, 0))],
        out_specs=pl.BlockSpec((BLK, Fn), lambda i: (i, 0)),
        out_shape=jax.ShapeDtypeStruct((NP, Fn), jnp.float32),
    )(parts, g, dinv, b, W)


def _pool_body(p_ref, g_ref, dinv_ref, b_ref, nb_ref, o_ref):
    i = pl.program_id(0)
    sfull = p_ref[0] + p_ref[1] + g_ref[...]
    h = jnp.maximum(sfull * dinv_ref[...] + b_ref[...], 0.0)
    contrib = jnp.sum(h * nb_ref[...], axis=0, keepdims=True)

    @pl.when(i == 0)
    def _():
        o_ref[...] = jnp.zeros_like(o_ref)

    o_ref[...] += contrib


def _pool_tc(parts, g, dinv, b, nb):
    return pl.pallas_call(
        _pool_body,
        grid=(NP // BLK,),
        in_specs=[pl.BlockSpec((2, BLK, F3), lambda i: (0, i, 0)),
                  pl.BlockSpec((BLK, F3), lambda i: (i, 0)),
                  pl.BlockSpec((BLK, 1), lambda i: (i, 0)),
                  pl.BlockSpec((1, F3), lambda i: (0, 0)),
                  pl.BlockSpec((BLK, 1), lambda i: (i, 0))],
        out_specs=pl.BlockSpec((1, F3), lambda i: (0, 0)),
        out_shape=jax.ShapeDtypeStruct((1, F3), jnp.float32),
    )(parts, g, dinv, b, nb)


def _head_body(pool_ref, wfc_ref, bfc_ref, wsc_ref, bsc_ref, o_ref):
    pooled = pool_ref[...] * (1.0 / N)
    fc = jnp.maximum(
        jnp.dot(pooled, wfc_ref[...], preferred_element_type=jnp.float32)
        + bfc_ref[...], 0.0)
    sc = (jnp.dot(fc, wsc_ref[...], preferred_element_type=jnp.float32)
          + bsc_ref[...])
    m = jnp.max(sc, axis=1, keepdims=True)
    z = sc - m
    o_ref[...] = z - jnp.log(jnp.sum(jnp.exp(z), axis=1, keepdims=True))


def _head_tc(pooled, Wfc, bfc, Wsc, bsc):
    return pl.pallas_call(
        _head_body,
        in_specs=[pl.BlockSpec((1, F3), lambda: (0, 0)),
                  pl.BlockSpec((F3, BN), lambda: (0, 0)),
                  pl.BlockSpec((1, BN), lambda: (0, 0)),
                  pl.BlockSpec((BN, NC_), lambda: (0, 0)),
                  pl.BlockSpec((1, NC_), lambda: (0, 0))],
        out_specs=pl.BlockSpec((1, NC_), lambda: (0, 0)),
        out_shape=jax.ShapeDtypeStruct((1, NC_), jnp.float32),
    )(pooled, Wfc, bfc, Wsc, bsc)


def kernel(adj, features, neighbors, W1, b1, W2, b2, W3, b3, Wfc, bfc, Wsc, bsc):
    src = adj[0].astype(jnp.int32)
    dst = adj[1].astype(jnp.int32)
    pad = EP - E
    srcp = jnp.concatenate([src, jnp.zeros((pad,), jnp.int32)])
    dstp = jnp.concatenate([dst, jnp.full((pad,), DUMMY, jnp.int32)])
    xp = jnp.zeros((NP, D), jnp.float32).at[:N].set(features)
    z1 = jnp.zeros((NP,), jnp.float32)
    z64 = jnp.zeros((NP, F1), jnp.float32)
    z32 = jnp.zeros((NP, F2), jnp.float32)
    z16 = jnp.zeros((NP, F3), jnp.float32)
    selfd = jnp.zeros((NP, 1), jnp.float32).at[:N].set(1.0)
    nbf = jnp.zeros((NP, 1), jnp.float32).at[:N, 0].set(
        neighbors.astype(jnp.float32))

    deg_parts = _deg_sc(dstp, z1)
    dinv = _dinv_tc(jnp.transpose(deg_parts), selfd)
    g1 = _mm1_tc(xp, W1, dinv)
    p1 = _agg64(g1, z64, srcp, dstp)
    g2 = _layer_tc(p1, g1, dinv, b1.reshape(1, -1), W2, F1, F2)
    p2 = _agg32(g2, z32, srcp, dstp)
    g3 = _layer_tc(p2, g2, dinv, b2.reshape(1, -1), W3, F2, F3)
    p3 = _agg16(g3, z16, srcp, dstp)
    pooled = _pool_tc(p3, g3, dinv, b3.reshape(1, -1), nbf)
    return _head_tc(pooled, Wfc, bfc.reshape(1, -1), Wsc, bsc.reshape(1, -1))


# speedup vs baseline: 16.8169x; 1.2867x over previous
import functools

import jax
import jax.numpy as jnp
from jax import lax
from jax.experimental import pallas as pl
from jax.experimental.pallas import tpu as pltpu
from jax.experimental.pallas import tpu_sc as plsc

N = 10000
NP = 10240
DUMMY = 10008
E = 320000
EP = 327680
CH = 128
PER_TILE = EP // 32
NCHUNK = PER_TILE // CH
NTILE = 16
RPT = NP // NTILE
NBUF = 4
NROUND = NCHUNK // NBUF
D = 128
F1, F2, F3, BN, NC_ = 64, 32, 16, 8, 10
BLK = 1024
NBLK = NP // BLK

_MESH = plsc.VectorSubcoreMesh(core_axis_name="c", subcore_axis_name="s")
_SC_PARAMS = pltpu.CompilerParams(use_tc_tiling_on_sc=False)


@functools.partial(
    pl.kernel,
    mesh=_MESH,
    compiler_params=_SC_PARAMS,
    out_type=jax.ShapeDtypeStruct((2, NP), jnp.float32),
    scratch_types=[
        pltpu.VMEM((NCHUNK, CH), jnp.int32),
        pltpu.VMEM((CH,), jnp.float32),
        pltpu.VMEM_SHARED((NP,), jnp.float32),
        pltpu.SemaphoreType.DMA,
    ],
)
def _deg_sc(dst_hbm, zero_hbm, out_hbm, idx_v, ones_v, acc_sh, sem):
    c = lax.axis_index("c")
    s = lax.axis_index("s")
    wid = c * NTILE + s
    for j in range(CH // 16):
        ones_v[pl.ds(j * 16, 16)] = jnp.ones((16,), jnp.float32)
    pltpu.sync_copy(dst_hbm.at[pl.ds(wid * NCHUNK, NCHUNK)], idx_v)
    pltpu.sync_copy(zero_hbm.at[pl.ds(s * RPT, RPT)],
                    acc_sh.at[pl.ds(s * RPT, RPT)])
    plsc.subcore_barrier()

    def round_body(r, carry):
        for b in range(NBUF):
            pltpu.async_copy(ones_v, acc_sh.at[idx_v.at[r * NBUF + b]], sem,
                             add=True)
        for b in range(NBUF):
            pltpu.make_async_copy(ones_v, acc_sh.at[idx_v.at[0]], sem).wait()
        return carry

    lax.fori_loop(0, NROUND, round_body, 0)
    plsc.subcore_barrier()
    pltpu.sync_copy(acc_sh.at[pl.ds(s * RPT, RPT)],
                    out_hbm.at[c, pl.ds(s * RPT, RPT)])


def _make_agg_sc(F):
    @functools.partial(
        pl.kernel,
        mesh=_MESH,
        compiler_params=_SC_PARAMS,
        out_type=jax.ShapeDtypeStruct((2, NP, F), jnp.float32),
        scratch_types=[
            pltpu.VMEM((NCHUNK, CH), jnp.int32),
            pltpu.VMEM((NCHUNK, CH), jnp.int32),
            [pltpu.VMEM((CH, F), jnp.float32)] * NBUF,
            pltpu.VMEM_SHARED((NP, F), jnp.float32),
            [pltpu.SemaphoreType.DMA] * NBUF,
            [pltpu.SemaphoreType.DMA] * NBUF,
        ],
    )
    def agg(g_hbm, zero_hbm, src_hbm, dst_hbm, out_hbm,
            src_v, dst_v, rows, acc_sh, gsem, ssem):
        c = lax.axis_index("c")
        s = lax.axis_index("s")
        wid = c * NTILE + s
        pltpu.sync_copy(src_hbm.at[pl.ds(wid * NCHUNK, NCHUNK)], src_v)
        pltpu.sync_copy(dst_hbm.at[pl.ds(wid * NCHUNK, NCHUNK)], dst_v)
        pltpu.sync_copy(zero_hbm.at[pl.ds(s * RPT, RPT)],
                        acc_sh.at[pl.ds(s * RPT, RPT)])
        plsc.subcore_barrier()

        def chunk(i, b, wait_scatter):
            if wait_scatter:
                pltpu.make_async_copy(rows[b], acc_sh.at[dst_v.at[0]],
                                      ssem[b]).wait()
            pltpu.async_copy(g_hbm.at[src_v.at[i]], rows[b], gsem[b]).wait()
            pltpu.async_copy(rows[b], acc_sh.at[dst_v.at[i]], ssem[b],
                             add=True)

        for b in range(NBUF):
            chunk(b, b, False)

        def round_body(r, carry):
            for b in range(NBUF):
                chunk(r * NBUF + b, b, True)
            return carry

        lax.fori_loop(1, NROUND, round_body, 0)
        for b in range(NBUF):
            pltpu.make_async_copy(rows[b], acc_sh.at[dst_v.at[0]],
                                  ssem[b]).wait()
        plsc.subcore_barrier()
        pltpu.sync_copy(acc_sh.at[pl.ds(s * RPT, RPT)],
                        out_hbm.at[c, pl.ds(s * RPT, RPT)])

    return agg


_agg64 = _make_agg_sc(F1)
_agg32 = _make_agg_sc(F2)
_agg16 = _make_agg_sc(F3)


def _mm1_body(degT_ref, selfd_ref, x_ref, w_ref, g_ref, dinv_ref):
    d = degT_ref[:, 0:1] + degT_ref[:, 1:2] + selfd_ref[...]
    dinv = jnp.where(d > 0, lax.rsqrt(jnp.maximum(d, 1e-12)),
                     jnp.zeros_like(d))
    dinv_ref[...] = dinv
    h = jnp.dot(x_ref[...], w_ref[...], preferred_element_type=jnp.float32)
    g_ref[...] = h * dinv


def _mm1_tc(degT, selfd, xp, W1):
    return pl.pallas_call(
        _mm1_body,
        grid=(NBLK,),
        in_specs=[pl.BlockSpec((BLK, 2), lambda i: (i, 0)),
                  pl.BlockSpec((BLK, 1), lambda i: (i, 0)),
                  pl.BlockSpec((BLK, D), lambda i: (i, 0)),
                  pl.BlockSpec((D, F1), lambda i: (0, 0))],
        out_specs=[pl.BlockSpec((BLK, F1), lambda i: (i, 0)),
                   pl.BlockSpec((BLK, 1), lambda i: (i, 0))],
        out_shape=[jax.ShapeDtypeStruct((NP, F1), jnp.float32),
                   jax.ShapeDtypeStruct((NP, 1), jnp.float32)],
    )(degT, selfd, xp, W1)


def _layer_body(p_ref, g_ref, dinv_ref, b_ref, w_ref, o_ref):
    sfull = p_ref[0] + p_ref[1] + g_ref[...]
    h = jnp.maximum(sfull * dinv_ref[...] + b_ref[...], 0.0)
    o_ref[...] = jnp.dot(h, w_ref[...],
                         preferred_element_type=jnp.float32) * dinv_ref[...]


def _layer_tc(parts, g, dinv, b, W, F, Fn):
    return pl.pallas_call(
        _layer_body,
        grid=(NBLK,),
        in_specs=[pl.BlockSpec((2, BLK, F), lambda i: (0, i, 0)),
                  pl.BlockSpec((BLK, F), lambda i: (i, 0)),
                  pl.BlockSpec((BLK, 1), lambda i: (i, 0)),
                  pl.BlockSpec((1, F), lambda i: (0, 0)),
                  pl.BlockSpec((F, Fn), lambda i: (0, 0))],
        out_specs=pl.BlockSpec((BLK, Fn), lambda i: (i, 0)),
        out_shape=jax.ShapeDtypeStruct((NP, Fn), jnp.float32),
    )(parts, g, dinv, b, W)


def _pool_head_body(p_ref, g_ref, dinv_ref, b_ref, nb_ref,
                    wfc_ref, bfc_ref, wsc_ref, bsc_ref,
                    pool_ref, o_ref):
    i = pl.program_id(0)
    sfull = p_ref[0] + p_ref[1] + g_ref[...]
    h = jnp.maximum(sfull * dinv_ref[...] + b_ref[...], 0.0)
    contrib = jnp.sum(h * nb_ref[...], axis=0, keepdims=True)

    @pl.when(i == 0)
    def _():
        pool_ref[...] = jnp.zeros_like(pool_ref)

    pool_ref[...] += contrib

    @pl.when(i == NBLK - 1)
    def _():
        pooled = pool_ref[...] * (1.0 / N)
        fc = jnp.maximum(
            jnp.dot(pooled, wfc_ref[...], preferred_element_type=jnp.float32)
            + bfc_ref[...], 0.0)
        sc = (jnp.dot(fc, wsc_ref[...], preferred_element_type=jnp.float32)
              + bsc_ref[...])
        m = jnp.max(sc, axis=1, keepdims=True)
        z = sc - m
        o_ref[...] = z - jnp.log(jnp.sum(jnp.exp(z), axis=1, keepdims=True))


def _pool_head_tc(parts, g, dinv, b, nb, Wfc, bfc, Wsc, bsc):
    _, out = pl.pallas_call(
        _pool_head_body,
        grid=(NBLK,),
        in_specs=[pl.BlockSpec((2, BLK, F3), lambda i: (0, i, 0)),
                  pl.BlockSpec((BLK, F3), lambda i: (i, 0)),
                  pl.BlockSpec((BLK, 1), lambda i: (i, 0)),
                  pl.BlockSpec((1, F3), lambda i: (0, 0)),
                  pl.BlockSpec((BLK, 1), lambda i: (i, 0)),
                  pl.BlockSpec((F3, BN), lambda i: (0, 0)),
                  pl.BlockSpec((1, BN), lambda i: (0, 0)),
                  pl.BlockSpec((BN, NC_), lambda i: (0, 0)),
                  pl.BlockSpec((1, NC_), lambda i: (0, 0))],
        out_specs=[pl.BlockSpec((1, F3), lambda i: (0, 0)),
                   pl.BlockSpec((1, NC_), lambda i: (0, 0))],
        out_shape=[jax.ShapeDtypeStruct((1, F3), jnp.float32),
                   jax.ShapeDtypeStruct((1, NC_), jnp.float32)],
    )(parts, g, dinv, b, nb, Wfc, bfc, Wsc, bsc)
    return out


def kernel(adj, features, neighbors, W1, b1, W2, b2, W3, b3, Wfc, bfc, Wsc, bsc):
    src = adj[0].astype(jnp.int32)
    dst = adj[1].astype(jnp.int32)
    pad = EP - E
    srcp = jnp.concatenate([src, jnp.zeros((pad,), jnp.int32)])
    dstp = jnp.concatenate([dst, jnp.full((pad,), DUMMY, jnp.int32)])
    src2 = srcp.reshape(EP // CH, CH)
    dst2 = dstp.reshape(EP // CH, CH)
    xp = jnp.zeros((NP, D), jnp.float32).at[:N].set(features)
    z1 = jnp.zeros((NP,), jnp.float32)
    z64 = jnp.zeros((NP, F1), jnp.float32)
    z32 = jnp.zeros((NP, F2), jnp.float32)
    z16 = jnp.zeros((NP, F3), jnp.float32)
    selfd = jnp.zeros((NP, 1), jnp.float32).at[:N].set(1.0)
    nbf = jnp.zeros((NP, 1), jnp.float32).at[:N, 0].set(
        neighbors.astype(jnp.float32))

    deg_parts = _deg_sc(dst2, z1)
    g1, dinv = _mm1_tc(jnp.transpose(deg_parts), selfd, xp, W1)
    p1 = _agg64(g1, z64, src2, dst2)
    g2 = _layer_tc(p1, g1, dinv, b1.reshape(1, -1), W2, F1, F2)
    p2 = _agg32(g2, z32, src2, dst2)
    g3 = _layer_tc(p2, g2, dinv, b2.reshape(1, -1), W3, F2, F3)
    p3 = _agg16(g3, z16, src2, dst2)
    return _pool_head_tc(p3, g3, dinv, b3.reshape(1, -1), nbf,
                         Wfc, bfc.reshape(1, -1), Wsc, bsc.reshape(1, -1))
